# Initial kernel scaffold; baseline (speedup 1.0000x reference)
import jax
import jax.numpy as jnp
from jax.experimental import pallas as pl


def kernel(feat, feat_mask, edge_mask, W, edge_index, x, y):
    raise NotImplementedError("write your pallas kernel here")


# speedup vs baseline: 48.8730x; 48.8730x over previous
import functools

import jax
import jax.numpy as jnp
from jax import lax
from jax.experimental import pallas as pl
from jax.experimental.pallas import tpu as pltpu
from jax.experimental.pallas import tpu_sc as plsc

N = 10000
E = 320000
D = 128
ALPHA1 = 0.005
ALPHA2 = 1.0
BETA1 = 1.0
BETA2 = 0.1
EPS = 1e-15

_INFO = plsc.get_sparse_core_info()
NC = _INFO.num_cores
NS = _INFO.num_subcores
L = _INFO.num_lanes
NW = NC * NS
EW = E // NW
NV = EW // L
GROUP = 25
NG = NV // GROUP
ACC = 4 * D

assert E % NW == 0 and EW % L == 0 and NV % GROUP == 0

_PIB = jax.lax.GatherScatterMode.PROMISE_IN_BOUNDS


_GDN = lax.GatherDimensionNumbers(
    offset_dims=(), collapsed_slice_dims=(0,), start_index_map=(0,))


def _splat(vec, j):
    idx = jnp.full((L, 1), j, jnp.int32)
    return lax.gather(vec, idx, _GDN, slice_sizes=(1,), mode=_PIB)


def _sc_scan_kernel(dst_hbm, src_hbm, em_hbm, feat_hbm, xv_hbm, yv_hbm,
                    out_hbm, dstv, xv, yv, s16, e16, rows, accv, sem):
    wid = lax.axis_index("s") * NC + lax.axis_index("c")
    base = wid * EW
    pltpu.sync_copy(dst_hbm.at[pl.ds(base, EW)], dstv)
    pltpu.sync_copy(xv_hbm, xv)
    pltpu.sync_copy(yv_hbm, yv)
    xvec = xv[...]
    yvec = yv[...]
    thr = jnp.max(jnp.maximum(xvec, yvec))

    for i in range(ACC // L):
        accv[pl.ds(i * L, L)] = jnp.zeros((L,), jnp.float32)

    def handle_vec(off):
        pltpu.sync_copy(src_hbm.at[pl.ds(base + off, L)], s16)
        pltpu.sync_copy(em_hbm.at[pl.ds(base + off, L)], e16)
        pltpu.async_copy(feat_hbm.at[s16], rows, sem).wait()
        d16 = dstv[pl.ds(off, L)]
        e16v = e16[...]
        wfx = (d16 == xvec).astype(jnp.float32)
        wfy = (d16 == yvec).astype(jnp.float32)
        kp = (e16v >= 0.0).astype(jnp.float32)
        wmx = wfx * kp
        wmy = wfy * kp
        parts = [[jnp.zeros((L,), jnp.float32) for _ in range(D // L)]
                 for _ in range(4)]
        for j in range(L):
            ws = (_splat(wfx, j), _splat(wfy, j), _splat(wmx, j), _splat(wmy, j))
            for b in range(D // L):
                row = rows[j, pl.ds(b * L, L)]
                for k in range(4):
                    parts[k][b] = parts[k][b] + ws[k] * row
        for k in range(4):
            for b in range(D // L):
                o = k * D + b * L
                accv[pl.ds(o, L)] = accv[pl.ds(o, L)] + parts[k][b]

    def fine_body(j, goff):
        off = goff + j * L
        d16 = dstv[pl.ds(off, L)]
        hit = (d16 == xvec) | (d16 == yvec)
        cnt = jnp.max(hit.astype(jnp.int32))

        @pl.when(cnt > 0)
        def _():
            handle_vec(off)

        return goff

    def group_body(g, _):
        goff = g * (GROUP * L)

        def min_body(j, mn):
            return jnp.minimum(mn, dstv[pl.ds(goff + j * L, L)])

        mn = lax.fori_loop(0, GROUP, min_body,
                           jnp.full((L,), jnp.iinfo(jnp.int32).max, jnp.int32))
        gmin = jnp.min(mn)

        @pl.when(gmin <= thr)
        def _():
            lax.fori_loop(0, GROUP, fine_body, goff)

        return 0

    lax.fori_loop(0, NG, group_body, 0)
    pltpu.sync_copy(accv, out_hbm.at[pl.ds(wid * ACC, ACC)])


def _sc_scan(dst, src, em, feat, xv, yv):
    mesh = plsc.VectorSubcoreMesh(core_axis_name="c", subcore_axis_name="s")
    f = functools.partial(
        pl.kernel,
        mesh=mesh,
        compiler_params=pltpu.CompilerParams(needs_layout_passes=False),
        out_type=jax.ShapeDtypeStruct((NW * ACC,), jnp.float32),
        scratch_types=[
            pltpu.VMEM((EW,), jnp.int32),
            pltpu.VMEM((L,), jnp.int32),
            pltpu.VMEM((L,), jnp.int32),
            pltpu.VMEM((L,), jnp.int32),
            pltpu.VMEM((L,), jnp.float32),
            pltpu.VMEM((L, D), jnp.float32),
            pltpu.VMEM((ACC,), jnp.float32),
            pltpu.SemaphoreType.DMA,
        ],
    )(_sc_scan_kernel)
    return f(dst, src, em, feat, xv, yv)


def _tc_finish_kernel(em_ref, fm_ref, w_ref, part_ref, out_ref):
    m = em_ref[...]
    em = jax.nn.sigmoid(m)
    ent_e = -em * jnp.log(em + EPS) - (1.0 - em) * jnp.log(1.0 - em + EPS)
    fm = jax.nn.sigmoid(fm_ref[...])
    ent_f = -fm * jnp.log(fm + EPS) - (1.0 - fm) * jnp.log(1.0 - fm + EPS)
    s = jnp.sum(part_ref[...], axis=0)
    rsel = (lax.broadcasted_iota(jnp.int32, (4, 1), 0) >= 2).astype(jnp.float32)
    scale = 1.0 + rsel * (fm - 1.0)
    logits = jnp.dot(s * scale, w_ref[...],
                     preferred_element_type=jnp.float32)
    pred = jnp.sum(logits[0:1] * logits[1:2])
    lp = jnp.sum(logits[2:3] * logits[3:4])
    loss = (lp - pred
            + ALPHA1 * jnp.sum(em)
            + ALPHA2 * (jnp.sum(ent_e) / E)
            + BETA1 * jnp.mean(fm)
            + BETA2 * jnp.mean(ent_f))
    out_ref[0, 0] = loss


def _tc_finish(em2d, feat_mask, W, partials):
    return pl.pallas_call(
        _tc_finish_kernel,
        out_shape=jax.ShapeDtypeStruct((1, 1), jnp.float32),
        out_specs=pl.BlockSpec(memory_space=pltpu.SMEM),
    )(em2d, feat_mask, W, partials)


def kernel(feat, feat_mask, edge_mask, W, edge_index, x, y):
    src = edge_index[0]
    dst = edge_index[1]
    xv = jnp.full((L,), x, jnp.int32)
    yv = jnp.full((L,), y, jnp.int32)
    partials = _sc_scan(dst, src, edge_mask, feat, xv, yv)
    em2d = edge_mask.reshape(E // D, D)
    loss = _tc_finish(em2d, feat_mask, W, partials.reshape(NW, 4, D))
    return loss[0, 0]


# speedup vs baseline: 61.7397x; 1.2633x over previous
import functools

import jax
import jax.numpy as jnp
from jax import lax
from jax.experimental import pallas as pl
from jax.experimental.pallas import tpu as pltpu
from jax.experimental.pallas import tpu_sc as plsc

N = 10000
E = 320000
D = 128
ALPHA1 = 0.005
ALPHA2 = 1.0
BETA1 = 1.0
BETA2 = 0.1
EPS = 1e-15

_INFO = plsc.get_sparse_core_info()
NC = _INFO.num_cores
NS = _INFO.num_subcores
L = _INFO.num_lanes
NW = NC * NS
EW = E // NW
NV = EW // L
GROUP = 25
NG = NV // GROUP
ACC = 4 * D

assert E % NW == 0 and EW % L == 0 and NV % GROUP == 0

_PIB = jax.lax.GatherScatterMode.PROMISE_IN_BOUNDS


_GDN = lax.GatherDimensionNumbers(
    offset_dims=(), collapsed_slice_dims=(0,), start_index_map=(0,))


def _splat(vec, j):
    idx = jnp.full((L, 1), j, jnp.int32)
    return lax.gather(vec, idx, _GDN, slice_sizes=(1,), mode=_PIB)


def _sc_scan_kernel(ei_hbm, em_hbm, feat_hbm, xv_hbm, yv_hbm,
                    out_hbm, dstv, xv, yv, s16, e16, rows, accv, sem):
    wid = lax.axis_index("s") * NC + lax.axis_index("c")
    base = wid * EW
    pltpu.sync_copy(ei_hbm.at[pl.ds(E + base, EW)], dstv)
    pltpu.sync_copy(xv_hbm, xv)
    pltpu.sync_copy(yv_hbm, yv)
    xvec = xv[...]
    yvec = yv[...]
    thr = jnp.max(jnp.maximum(xvec, yvec))

    for i in range(ACC // L):
        accv[pl.ds(i * L, L)] = jnp.zeros((L,), jnp.float32)

    def handle_vec(off):
        pltpu.sync_copy(ei_hbm.at[pl.ds(base + off, L)], s16)
        pltpu.sync_copy(em_hbm.at[pl.ds(base + off, L)], e16)
        pltpu.async_copy(feat_hbm.at[s16], rows, sem).wait()
        d16 = dstv[pl.ds(off, L)]
        e16v = e16[...]
        wfx = (d16 == xvec).astype(jnp.float32)
        wfy = (d16 == yvec).astype(jnp.float32)
        kp = (e16v >= 0.0).astype(jnp.float32)
        wmx = wfx * kp
        wmy = wfy * kp
        parts = [[jnp.zeros((L,), jnp.float32) for _ in range(D // L)]
                 for _ in range(4)]
        for j in range(L):
            ws = (_splat(wfx, j), _splat(wfy, j), _splat(wmx, j), _splat(wmy, j))
            for b in range(D // L):
                row = rows[j, pl.ds(b * L, L)]
                for k in range(4):
                    parts[k][b] = parts[k][b] + ws[k] * row
        for k in range(4):
            for b in range(D // L):
                o = k * D + b * L
                accv[pl.ds(o, L)] = accv[pl.ds(o, L)] + parts[k][b]

    def fine_body(j, goff):
        off = goff + j * L
        d16 = dstv[pl.ds(off, L)]
        hit = (d16 == xvec) | (d16 == yvec)
        cnt = jnp.max(hit.astype(jnp.int32))

        @pl.when(cnt > 0)
        def _():
            handle_vec(off)

        return goff

    def group_body(g, _):
        goff = g * (GROUP * L)
        mn = dstv[pl.ds(goff, L)]
        for j in range(1, GROUP):
            mn = jnp.minimum(mn, dstv[pl.ds(goff + j * L, L)])
        gmin = jnp.min(mn)

        @pl.when(gmin <= thr)
        def _():
            lax.fori_loop(0, GROUP, fine_body, goff)

        return 0

    lax.fori_loop(0, NG, group_body, 0)
    pltpu.sync_copy(accv, out_hbm.at[pl.ds(wid * ACC, ACC)])


def _sc_scan(ei, em, feat, xv, yv):
    mesh = plsc.VectorSubcoreMesh(core_axis_name="c", subcore_axis_name="s")
    f = functools.partial(
        pl.kernel,
        mesh=mesh,
        compiler_params=pltpu.CompilerParams(needs_layout_passes=False),
        out_type=jax.ShapeDtypeStruct((NW * ACC,), jnp.float32),
        scratch_types=[
            pltpu.VMEM((EW,), jnp.int32),
            pltpu.VMEM((L,), jnp.int32),
            pltpu.VMEM((L,), jnp.int32),
            pltpu.VMEM((L,), jnp.int32),
            pltpu.VMEM((L,), jnp.float32),
            pltpu.VMEM((L, D), jnp.float32),
            pltpu.VMEM((ACC,), jnp.float32),
            pltpu.SemaphoreType.DMA,
        ],
    )(_sc_scan_kernel)
    return f(ei, em, feat, xv, yv)


def _tc_finish_kernel(em_ref, fm_ref, w_ref, part_ref, out_ref):
    m = em_ref[...]
    em = jax.nn.sigmoid(m)
    ent_e = -em * jnp.log(em + EPS) - (1.0 - em) * jnp.log(1.0 - em + EPS)
    fm = jax.nn.sigmoid(fm_ref[...])
    ent_f = -fm * jnp.log(fm + EPS) - (1.0 - fm) * jnp.log(1.0 - fm + EPS)
    s = jnp.sum(part_ref[...], axis=0)
    rsel = (lax.broadcasted_iota(jnp.int32, (4, 1), 0) >= 2).astype(jnp.float32)
    scale = 1.0 + rsel * (fm - 1.0)
    logits = jnp.dot(s * scale, w_ref[...],
                     preferred_element_type=jnp.float32)
    pred = jnp.sum(logits[0:1] * logits[1:2])
    lp = jnp.sum(logits[2:3] * logits[3:4])
    loss = (lp - pred
            + ALPHA1 * jnp.sum(em)
            + ALPHA2 * (jnp.sum(ent_e) / E)
            + BETA1 * jnp.mean(fm)
            + BETA2 * jnp.mean(ent_f))
    out_ref[0, 0] = loss


def _tc_finish(em2d, feat_mask, W, partials):
    return pl.pallas_call(
        _tc_finish_kernel,
        out_shape=jax.ShapeDtypeStruct((1, 1), jnp.float32),
        out_specs=pl.BlockSpec(memory_space=pltpu.SMEM),
    )(em2d, feat_mask, W, partials)


def kernel(feat, feat_mask, edge_mask, W, edge_index, x, y):
    xv = jnp.full((L,), x, jnp.int32)
    yv = jnp.full((L,), y, jnp.int32)
    partials = _sc_scan(edge_index.reshape(2 * E), edge_mask, feat, xv, yv)
    em2d = edge_mask.reshape(E // D, D)
    loss = _tc_finish(em2d, feat_mask, W, partials.reshape(NW, 4, D))
    return loss[0, 0]
